# baseline (device time: 75702 ns/iter reference)
import jax
import jax.numpy as jnp
from jax import lax
from jax.experimental import pallas as pl
from jax.experimental.pallas import tpu as pltpu

N_DEV = 4
NKW = 4
NTO = 4
NC = 2
NL = 3


def kernel(x, Win0, Wout0, Win1, Wout1, Win2, Wout2):
    B, D = x.shape
    H = Win0.shape[1]
    DK = D // NKW
    TH = H // NTO
    DC = D // NC
    Bq = B // N_DEV

    def body(x_ref, win0, wout0, win1, wout1, win2, wout2, out_ref,
             xbuf, ybuf, hbuf, ysend, win_vm, wout_vm, ar_recv, rs_recv,
             load_sems, ar_send_sems, ar_recv_sems, rs_send_sems, rs_recv_sems):
        me = lax.axis_index("i")
        wins = [win0, win1, win2]
        wouts = [wout0, wout1, wout2]

        barrier = pltpu.get_barrier_semaphore()
        for d in range(1, N_DEV):
            pl.semaphore_signal(
                barrier, inc=1,
                device_id=((me + d) % N_DEV,),
                device_id_type=pl.DeviceIdType.MESH,
            )
        pl.semaphore_wait(barrier, N_DEV - 1)

        def issue_win(wi):
            l, k = divmod(wi, NKW)
            slot = wi % 2
            c = pltpu.make_async_copy(
                wins[l].at[pl.ds(k * DK, DK), :], win_vm.at[slot],
                load_sems.at[0, slot])
            c.start()
            return c

        def issue_wout(oi):
            l, r = divmod(oi, NC * NTO)
            cc, t = divmod(r, NTO)
            slot = oi % 2
            c = pltpu.make_async_copy(
                wouts[l].at[pl.ds(t * TH, TH), pl.ds(cc * DC, DC)],
                wout_vm.at[slot], load_sems.at[1, slot])
            c.start()
            return c

        def ar_chunk_descs(l, cc):
            return [
                pltpu.make_async_remote_copy(
                    src_ref=ysend.at[:, pl.ds(cc * DC, DC)],
                    dst_ref=ar_recv.at[l, cc, d - 1],
                    send_sem=ar_send_sems.at[l, cc, d - 1],
                    recv_sem=ar_recv_sems.at[l, cc, d - 1],
                    device_id=((me + d) % N_DEV,),
                    device_id_type=pl.DeviceIdType.MESH,
                )
                for d in range(1, N_DEV)
            ]

        pend_w = {wi: issue_win(wi) for wi in range(2)}
        pend_o = {oi: issue_wout(oi) for oi in range(2)}
        ar = {}
        for l in range(NL):
            for k in range(NKW):
                if l > 0 and k * DK % DC == 0:
                    cc = (k * DK) // DC
                    for r in ar[(l - 1, cc)]:
                        r.wait_recv()
                    sl = pl.ds(cc * DC, DC)
                    xbuf[:, sl] = (
                        ybuf[:, sl]
                        + ar_recv[l - 1, cc, 0].astype(jnp.float32)
                        + ar_recv[l - 1, cc, 1].astype(jnp.float32)
                        + ar_recv[l - 1, cc, 2].astype(jnp.float32))
                wi = l * NKW + k
                c = pend_w.pop(wi)
                c.wait()
                xv = (x_ref if l == 0 else xbuf)[:, pl.ds(k * DK, DK)]
                hp = jnp.dot(xv, win_vm[wi % 2],
                             preferred_element_type=jnp.float32)
                if k == 0:
                    hbuf[...] = hp
                else:
                    hbuf[...] = hbuf[...] + hp
                if wi + 2 < NL * NKW:
                    pend_w[wi + 2] = issue_win(wi + 2)
            hbuf[...] = jnp.maximum(hbuf[...], 0.0)
            for cc in range(NC):
                ysl = pl.ds(cc * DC, DC)
                for t in range(NTO):
                    oi = l * NC * NTO + cc * NTO + t
                    c = pend_o.pop(oi)
                    c.wait()
                    hv = hbuf[:, pl.ds(t * TH, TH)]
                    yp = jnp.dot(hv, wout_vm[oi % 2],
                                 preferred_element_type=jnp.float32)
                    if t == 0:
                        ybuf[:, ysl] = yp
                    else:
                        ybuf[:, ysl] = ybuf[:, ysl] + yp
                    if oi + 2 < NL * NC * NTO:
                        pend_o[oi + 2] = issue_wout(oi + 2)
                if l > 0:
                    for r in ar[(l - 1, cc)]:
                        r.wait_send()
                ysend[:, ysl] = ybuf[:, ysl].astype(jnp.bfloat16)
                if l < NL - 1:
                    descs = ar_chunk_descs(l, cc)
                    for r in descs:
                        r.start()
                    ar[(l, cc)] = descs
                else:
                    rs = []
                    for d in range(1, N_DEV):
                        j = (me + d) % N_DEV
                        r = pltpu.make_async_remote_copy(
                            src_ref=ysend.at[pl.ds(j * Bq, Bq), ysl],
                            dst_ref=rs_recv.at[cc, d - 1],
                            send_sem=rs_send_sems.at[cc, d - 1],
                            recv_sem=rs_recv_sems.at[cc, d - 1],
                            device_id=(j,),
                            device_id_type=pl.DeviceIdType.MESH,
                        )
                        r.start()
                        rs.append(r)
                    ar[(l, cc)] = rs
        for cc in range(NC):
            for r in ar[(NL - 1, cc)]:
                r.wait_send()
            for r in ar[(NL - 1, cc)]:
                r.wait_recv()
            ysl = pl.ds(cc * DC, DC)
            out_ref[:, ysl] = (
                ybuf[pl.ds(me * Bq, Bq), ysl]
                + rs_recv[cc, 0].astype(jnp.float32)
                + rs_recv[cc, 1].astype(jnp.float32)
                + rs_recv[cc, 2].astype(jnp.float32))

    return pl.pallas_call(
        body,
        out_shape=jax.ShapeDtypeStruct((Bq, D), jnp.float32),
        in_specs=[
            pl.BlockSpec(memory_space=pltpu.MemorySpace.VMEM),
            pl.BlockSpec(memory_space=pl.ANY),
            pl.BlockSpec(memory_space=pl.ANY),
            pl.BlockSpec(memory_space=pl.ANY),
            pl.BlockSpec(memory_space=pl.ANY),
            pl.BlockSpec(memory_space=pl.ANY),
            pl.BlockSpec(memory_space=pl.ANY),
        ],
        out_specs=pl.BlockSpec(memory_space=pltpu.MemorySpace.VMEM),
        scratch_shapes=[
            pltpu.VMEM((B, D), jnp.float32),
            pltpu.VMEM((B, D), jnp.float32),
            pltpu.VMEM((B, H), jnp.float32),
            pltpu.VMEM((B, D), jnp.bfloat16),
            pltpu.VMEM((2, D // NKW, H), jnp.float32),
            pltpu.VMEM((2, H // NTO, D // NC), jnp.float32),
            pltpu.VMEM((2, NC, 3, B, D // NC), jnp.bfloat16),
            pltpu.VMEM((NC, 3, Bq, D // NC), jnp.bfloat16),
            pltpu.SemaphoreType.DMA((2, 2)),
            pltpu.SemaphoreType.DMA((2, NC, 3)),
            pltpu.SemaphoreType.DMA((2, NC, 3)),
            pltpu.SemaphoreType.DMA((NC, 3)),
            pltpu.SemaphoreType.DMA((NC, 3)),
        ],
        compiler_params=pltpu.CompilerParams(
            collective_id=0,
            vmem_limit_bytes=60 * 1024 * 1024,
        ),
    )(x, Win0, Wout0, Win1, Wout1, Win2, Wout2)
